# baseline (speedup 1.0000x reference)
import functools

import jax
import jax.numpy as jnp
from jax import lax
from jax.experimental import pallas as pl
from jax.experimental.pallas import tpu as pltpu
from jax.experimental.pallas import tpu_sc as plsc

_D = 64
_SPLITS = (0, 128)
_R = 4
_NBUF = 2


def _gather_kernel(NB, H):
    info = plsc.get_sparse_core_info()
    NW = info.num_cores * info.num_subcores
    rows_per_w = NB // NW
    chunks_per_w = rows_per_w // _R
    rounds = chunks_per_w // _NBUF
    lens = [_SPLITS[i + 1] - _SPLITS[i] for i in range(len(_SPLITS) - 1)]
    lens.append(H - _SPLITS[-1])

    mesh = plsc.VectorSubcoreMesh(core_axis_name="c", subcore_axis_name="s")

    scratch = (
        [pltpu.VMEM((_R, H), jnp.int32) for _ in range(_NBUF)]
        + [pltpu.VMEM((_R, H, _D), jnp.float32) for _ in range(_NBUF)]
        + [pltpu.SemaphoreType.DMA for _ in range(2 * _NBUF)]
    )

    @functools.partial(
        pl.kernel,
        mesh=mesh,
        out_type=jax.ShapeDtypeStruct((NB, H, _D), jnp.float32),
        scratch_types=scratch,
        compiler_params=pltpu.CompilerParams(use_tc_tiling_on_sc=False),
    )
    def k(table_hbm, idx_hbm, out_hbm, *bufs):
        idx_v = bufs[:_NBUF]
        rows_v = bufs[_NBUF:2 * _NBUF]
        sem_g = bufs[2 * _NBUF:3 * _NBUF]
        sem_w = bufs[3 * _NBUF:]

        wid = lax.axis_index("s") * info.num_cores + lax.axis_index("c")
        row_base = wid * rows_per_w

        def gather_copies(b):
            for r in range(_R):
                for s, ln in zip(_SPLITS, lens):
                    yield pltpu.make_async_copy(
                        table_hbm.at[idx_v[b].at[r, pl.ds(s, ln)]],
                        rows_v[b].at[r, pl.ds(s, ln), :],
                        sem_g[b],
                    )

        def fire_gather(b, chunk):
            pltpu.sync_copy(
                idx_hbm.at[pl.ds(row_base + chunk * _R, _R)], idx_v[b])
            for cp in gather_copies(b):
                cp.start()

        def wait_gather(b):
            for cp in gather_copies(b):
                cp.wait()

        def write_copy(b, chunk):
            return pltpu.make_async_copy(
                rows_v[b],
                out_hbm.at[pl.ds(row_base + chunk * _R, _R)],
                sem_w[b],
            )

        for b in range(_NBUF):
            fire_gather(b, b)

        def body(r, carry):
            c0 = r * _NBUF
            for b in range(_NBUF):
                wait_gather(b)
                write_copy(b, c0 + b).start()
            for b in range(_NBUF):
                write_copy(b, c0 + b).wait()
                fire_gather(b, c0 + _NBUF + b)
            return carry

        lax.fori_loop(0, rounds - 1, body, 0)

        c0 = (rounds - 1) * _NBUF
        for b in range(_NBUF):
            wait_gather(b)
            write_copy(b, c0 + b).start()
        for b in range(_NBUF):
            write_copy(b, c0 + b).wait()

    return k


def kernel(embs, step):
    nb, h = step.shape
    return _gather_kernel(nb, h)(embs, step)
